# baseline (speedup 1.0000x reference)
import functools

import jax
import jax.numpy as jnp
from jax import lax
from jax.experimental import pallas as pl
from jax.experimental.pallas import tpu as pltpu
from jax.experimental.pallas import tpu_sc as plsc

NC, NS, L = 2, 16, 16
NW = NC * NS
B, S, D = 4, 8192, 1024
N = B * S
V = 1000
DW = D // 2
RPW = N // NW
C = 16
NCHUNK = RPW // C
NBUF = 4

_mesh = plsc.VectorSubcoreMesh(core_axis_name="c", subcore_axis_name="s")


@functools.partial(
    pl.kernel,
    out_type=jax.ShapeDtypeStruct((N, D), jnp.float32),
    mesh=_mesh,
    scratch_types=[
        pltpu.VMEM((RPW,), jnp.int32),
        pltpu.VMEM((NBUF, C, D), jnp.float32),
        pltpu.VMEM((NBUF, C, DW), jnp.int32),
        pltpu.SemaphoreType.DMA((NBUF,)),
        pltpu.SemaphoreType.DMA((NBUF,)),
        pltpu.SemaphoreType.DMA((NBUF,)),
    ],
)
def _sc_add_lookup(ids_hbm, x_hbm, table_hbm, out_hbm,
                   idx_v, in_v, rows_v, gsem, isem, osem):
    wid = lax.axis_index("s") * NC + lax.axis_index("c")
    base = wid * RPW
    pltpu.sync_copy(ids_hbm.at[pl.ds(base, RPW)], idx_v)

    def start_gather(ci, b):
        pltpu.async_copy(table_hbm.at[idx_v.at[pl.ds(ci * C, C)]],
                         rows_v.at[b], gsem.at[b])

    def start_input(ci, b):
        pltpu.async_copy(x_hbm.at[pl.ds(base + ci * C, C)],
                         in_v.at[b], isem.at[b])

    def start_scatter(ci, b):
        pltpu.async_copy(in_v.at[b], out_hbm.at[pl.ds(base + ci * C, C)],
                         osem.at[b])

    def wait_scatter(ci, b):
        pltpu.make_async_copy(in_v.at[b],
                              out_hbm.at[pl.ds(base + ci * C, C)],
                              osem.at[b]).wait()

    for k in range(NBUF):
        start_gather(k, k)
    for k in range(NBUF - 1):
        start_input(k, k)

    @pl.loop(0, NCHUNK, step=NBUF)
    def _group(g):
        for b in range(NBUF):
            ci = g + b
            bm1 = (b - 1) % NBUF
            pltpu.make_async_copy(table_hbm.at[idx_v.at[pl.ds(ci * C, C)]],
                                  rows_v.at[b], gsem.at[b]).wait()
            pltpu.make_async_copy(x_hbm.at[pl.ds(base + ci * C, C)],
                                  in_v.at[b], isem.at[b]).wait()

            @pl.loop(0, C)
            def _row(r):
                for j in range(D // 32):
                    w = rows_v[b, r, pl.ds(j * 16, 16)]
                    lo = lax.bitcast_convert_type(w << 16, jnp.float32)
                    hi = lax.bitcast_convert_type(w & jnp.int32(-65536), jnp.float32)
                    plsc.addupdate(in_v.at[b, r, pl.ds(j * 32, 16)], lo)
                    plsc.addupdate(in_v.at[b, r, pl.ds(j * 32 + 16, 16)], hi)

            @pl.when(ci + NBUF < NCHUNK)
            def _():
                start_gather(ci + NBUF, b)

            start_scatter(ci, b)

            @pl.when(ci >= 1)
            def _():
                wait_scatter(ci - 1, bm1)

            @pl.when(ci + NBUF - 1 < NCHUNK)
            def _():
                start_input(ci + NBUF - 1, bm1)

    wait_scatter(NCHUNK - 1, (NCHUNK - 1) % NBUF)


def kernel(model_type_ids, input_embeddings, table):
    ids = model_type_ids.reshape(N).astype(jnp.int32)
    x = input_embeddings.reshape(N, D)
    t = table.astype(jnp.bfloat16).reshape(V, D // 32, 2, 16)
    t = t.transpose(0, 1, 3, 2).reshape(V, DW, 2)
    t_packed = jax.lax.bitcast_convert_type(t, jnp.int32)
    out = _sc_add_lookup(ids, x, t_packed)
    return out.reshape(B, S, D)
